# baseline (speedup 1.0000x reference)
import functools

import jax
import jax.numpy as jnp
from jax import lax
from jax.experimental import pallas as pl
from jax.experimental.pallas import tpu as pltpu
from jax.experimental.pallas import tpu_sc as plsc

N_FIELDS = 3
B = 16384
D = 64
LANES = 16
NC = 2
NS = 16
NW = NC * NS
BPW = B // NW
CHUNK = 128
NCHUNK = BPW // CHUNK
TROWS = 500000


def _newmf_body(it0_hbm, it1_hbm, it2_hbm, table_hbm, out_hbm,
                idx0, idx1, idx2, pidx0, pidx1, pidx2,
                r00, r01, r10, r11, r20, r21, out_v, sem0, sem1):
    items_hbm = (it0_hbm, it1_hbm, it2_hbm)
    idx_v = (idx0, idx1, idx2)
    pidx_v = (pidx0, pidx1, pidx2)
    rows_v = ((r00, r10, r20), (r01, r11, r21))
    sems = (sem0, sem1)
    wid = lax.axis_index("s") * NC + lax.axis_index("c")
    base = wid * BPW

    for f in range(N_FIELDS):
        pltpu.sync_copy(items_hbm[f].at[pl.ds(base, BPW)], idx_v[f])

    def pidx_body(t, carry):
        sl = pl.ds(t * LANES, LANES)
        for f in range(N_FIELDS):
            pidx_v[f][sl] = lax.shift_right_logical(idx_v[f][sl], 1)
        return carry

    lax.fori_loop(0, BPW // LANES, pidx_body, 0)

    def fire(j, pb):
        return [
            pltpu.async_copy(
                table_hbm.at[pidx_v[f].at[pl.ds(j * CHUNK, CHUNK)]],
                rows_v[pb][f],
                sems[pb],
            )
            for f in range(N_FIELDS)
        ]

    lane = lax.iota(jnp.int32, LANES)
    perms = [jnp.bitwise_xor(lane, 1 << t) for t in range(4)]
    masks = [lane == j for j in range(LANES)]
    dnums = lax.GatherDimensionNumbers(
        offset_dims=(), collapsed_slice_dims=(0,), start_index_map=(0,))

    def _shuffle(v, perm):
        return lax.gather(
            v, perm[:, None], dimension_numbers=dnums, slice_sizes=(1,),
            mode=lax.GatherScatterMode.PROMISE_IN_BOUNDS)

    copies = fire(0, 0)
    for j in range(NCHUNK):
        pb = j % 2
        nxt = fire(j + 1, 1 - pb) if j + 1 < NCHUNK else []
        for c in copies:
            c.wait()
        copies = nxt
        bufs = rows_v[pb]

        def grp_body(g, carry, _j=j, _bufs=bufs):
            vec = jnp.zeros((LANES,), jnp.float32)
            gsl = pl.ds(_j * CHUNK + g * LANES, LANES)
            hoff = [(idx_v[f][gsl] & 1) * D for f in range(N_FIELDS)]
            for jj in range(LANES):
                r = g * LANES + jj
                acc = None
                offs = [hoff[f][jj] for f in range(N_FIELDS)]
                for k in range(D // LANES):
                    p = None
                    for f in range(N_FIELDS):
                        v = _bufs[f][r, pl.ds(offs[f] + k * LANES, LANES)]
                        p = v if p is None else p * v
                    acc = p if acc is None else acc + p
                for t in range(4):
                    acc = acc + _shuffle(acc, perms[t])
                vec = jnp.where(masks[jj], acc, vec)
            out_v[pl.ds(_j * CHUNK + g * LANES, LANES)] = (
                1.0 / (1.0 + jnp.exp(-vec)))
            return carry

        lax.fori_loop(0, CHUNK // LANES, grp_body, 0)

    pltpu.sync_copy(out_v, out_hbm.at[pl.ds(base, BPW)])


@functools.partial(
    pl.kernel,
    mesh=plsc.VectorSubcoreMesh(core_axis_name="c", subcore_axis_name="s"),
    out_type=jax.ShapeDtypeStruct((B,), jnp.float32),
    scratch_types=[
        pltpu.VMEM((BPW,), jnp.int32),
        pltpu.VMEM((BPW,), jnp.int32),
        pltpu.VMEM((BPW,), jnp.int32),
        pltpu.VMEM((BPW,), jnp.int32),
        pltpu.VMEM((BPW,), jnp.int32),
        pltpu.VMEM((BPW,), jnp.int32),
        pltpu.VMEM((CHUNK, 2 * D), jnp.float32),
        pltpu.VMEM((CHUNK, 2 * D), jnp.float32),
        pltpu.VMEM((CHUNK, 2 * D), jnp.float32),
        pltpu.VMEM((CHUNK, 2 * D), jnp.float32),
        pltpu.VMEM((CHUNK, 2 * D), jnp.float32),
        pltpu.VMEM((CHUNK, 2 * D), jnp.float32),
        pltpu.VMEM((BPW,), jnp.float32),
        pltpu.SemaphoreType.DMA,
        pltpu.SemaphoreType.DMA,
    ],
)
def _newmf(it0_hbm, it1_hbm, it2_hbm, table_hbm, out_hbm,
           idx0, idx1, idx2, pidx0, pidx1, pidx2,
           r00, r01, r10, r11, r20, r21, out_v, sem0, sem1):
    _newmf_body(it0_hbm, it1_hbm, it2_hbm, table_hbm, out_hbm,
                idx0, idx1, idx2, pidx0, pidx1, pidx2,
                r00, r01, r10, r11, r20, r21, out_v, sem0, sem1)


def kernel(items, item_table):
    table2 = item_table.reshape(TROWS, 2 * D)
    return _newmf(items[0], items[1], items[2], table2)


# speedup vs baseline: 1.6798x; 1.6798x over previous
import functools

import jax
import jax.numpy as jnp
from jax import lax
from jax.experimental import pallas as pl
from jax.experimental.pallas import tpu as pltpu
from jax.experimental.pallas import tpu_sc as plsc

N_FIELDS = 3
B = 16384
D = 64
LANES = 16
NC = 2
NS = 16
NW = NC * NS
BPW = B // NW
CHUNK = 128
NCHUNK = BPW // CHUNK


def _newmf_body(it0_hbm, it1_hbm, it2_hbm, table_hbm, out_hbm,
                idx0, idx1, idx2,
                r00, r01, r10, r11, r20, r21, out_v, sem0, sem1):
    items_hbm = (it0_hbm, it1_hbm, it2_hbm)
    idx_v = (idx0, idx1, idx2)
    rows_v = ((r00, r10, r20), (r01, r11, r21))
    sems = (sem0, sem1)
    wid = lax.axis_index("s") * NC + lax.axis_index("c")
    base = wid * BPW

    for f in range(N_FIELDS):
        pltpu.sync_copy(items_hbm[f].at[pl.ds(base, BPW)], idx_v[f])

    def fire(j, pb):
        def g_body(g, carry):
            iv = [idx_v[f][pl.ds(j * CHUNK + g * LANES, LANES)]
                  for f in range(N_FIELDS)]
            for jj in range(LANES):
                r = g * LANES + jj
                for f in range(N_FIELDS):
                    pltpu.async_copy(
                        table_hbm.at[pl.ds(iv[f][jj], 1)],
                        rows_v[pb][f].at[pl.ds(r, 1)],
                        sems[pb],
                    )
            return carry

        lax.fori_loop(0, CHUNK // LANES, g_body, 0)

    def drain(pb):
        for f in range(N_FIELDS):
            pltpu.make_async_copy(
                table_hbm.at[pl.ds(0, CHUNK)], rows_v[pb][f], sems[pb]
            ).wait()

    lane = lax.iota(jnp.int32, LANES)
    perms = [jnp.bitwise_xor(lane, 1 << t) for t in range(4)]
    masks = [lane == j for j in range(LANES)]
    dnums = lax.GatherDimensionNumbers(
        offset_dims=(), collapsed_slice_dims=(0,), start_index_map=(0,))

    def _shuffle(v, perm):
        return lax.gather(
            v, perm[:, None], dimension_numbers=dnums, slice_sizes=(1,),
            mode=lax.GatherScatterMode.PROMISE_IN_BOUNDS)

    fire(0, 0)
    for j in range(NCHUNK):
        pb = j % 2
        if j + 1 < NCHUNK:
            fire(j + 1, 1 - pb)
        drain(pb)
        bufs = rows_v[pb]

        def grp_body(g, carry, _j=j, _bufs=bufs):
            vec = jnp.zeros((LANES,), jnp.float32)
            for jj in range(LANES):
                r = g * LANES + jj
                acc = None
                for k in range(D // LANES):
                    sl = pl.ds(k * LANES, LANES)
                    p = _bufs[0][r, sl] * _bufs[1][r, sl] * _bufs[2][r, sl]
                    acc = p if acc is None else acc + p
                for t in range(4):
                    acc = acc + _shuffle(acc, perms[t])
                vec = jnp.where(masks[jj], acc, vec)
            out_v[pl.ds(_j * CHUNK + g * LANES, LANES)] = (
                1.0 / (1.0 + jnp.exp(-vec)))
            return carry

        lax.fori_loop(0, CHUNK // LANES, grp_body, 0)

    pltpu.sync_copy(out_v, out_hbm.at[pl.ds(base, BPW)])


@functools.partial(
    pl.kernel,
    mesh=plsc.VectorSubcoreMesh(core_axis_name="c", subcore_axis_name="s"),
    out_type=jax.ShapeDtypeStruct((B,), jnp.float32),
    scratch_types=[
        pltpu.VMEM((BPW,), jnp.int32),
        pltpu.VMEM((BPW,), jnp.int32),
        pltpu.VMEM((BPW,), jnp.int32),
        pltpu.VMEM((CHUNK, D), jnp.float32),
        pltpu.VMEM((CHUNK, D), jnp.float32),
        pltpu.VMEM((CHUNK, D), jnp.float32),
        pltpu.VMEM((CHUNK, D), jnp.float32),
        pltpu.VMEM((CHUNK, D), jnp.float32),
        pltpu.VMEM((CHUNK, D), jnp.float32),
        pltpu.VMEM((BPW,), jnp.float32),
        pltpu.SemaphoreType.DMA,
        pltpu.SemaphoreType.DMA,
    ],
)
def _newmf(it0_hbm, it1_hbm, it2_hbm, table_hbm, out_hbm,
           idx0, idx1, idx2,
           r00, r01, r10, r11, r20, r21, out_v, sem0, sem1):
    _newmf_body(it0_hbm, it1_hbm, it2_hbm, table_hbm, out_hbm,
                idx0, idx1, idx2,
                r00, r01, r10, r11, r20, r21, out_v, sem0, sem1)


def kernel(items, item_table):
    return _newmf(items[0], items[1], items[2], item_table)
